# initial kernel scaffold (unmeasured)
import jax
import jax.numpy as jnp
from jax import lax
from jax.experimental import pallas as pl
from jax.experimental.pallas import tpu as pltpu

N_DEV = 4


def kernel(x, w_mat, scale_x, scale_w):
    m, k_shard = x.shape
    _, n = w_mat.shape
    m_blk = m // N_DEV

    def body(x_ref, w_ref, sx_ref, sw_ref, out_ref, comm_ref,
             rs_send_sems, rs_recv_sems, ag_send_sems, ag_recv_sems):
        my = lax.axis_index("i")
        left = (my - 1) % N_DEV
        right = (my + 1) % N_DEV

        barrier_sem = pltpu.get_barrier_semaphore()
        for nbr in [left, right]:
            pl.semaphore_signal(
                barrier_sem, inc=1,
                device_id=(nbr,), device_id_type=pl.DeviceIdType.MESH,
            )
        pl.semaphore_wait(barrier_sem, 2)

        xb = x_ref[:, :].astype(jnp.bfloat16)
        wb = w_ref[:, :].astype(jnp.bfloat16)
        out_ref[:, :] = jnp.dot(xb, wb, preferred_element_type=jnp.float32)

        for s in range(N_DEV - 1):
            send_b = (my - 1 - s) % N_DEV
            recv_b = (my - 2 - s) % N_DEV
            rdma = pltpu.make_async_remote_copy(
                src_ref=out_ref.at[pl.ds(send_b * m_blk, m_blk), :],
                dst_ref=comm_ref.at[s],
                send_sem=rs_send_sems.at[s],
                recv_sem=rs_recv_sems.at[s],
                device_id=(right,),
                device_id_type=pl.DeviceIdType.MESH,
            )
            rdma.start()
            rdma.wait()
            out_ref[pl.ds(recv_b * m_blk, m_blk), :] = (
                out_ref[pl.ds(recv_b * m_blk, m_blk), :] + comm_ref[s]
            )

        scale = sx_ref[0] * sw_ref[0]
        my_rows = pl.ds(my * m_blk, m_blk)
        out_ref[my_rows, :] = jnp.maximum(out_ref[my_rows, :] * scale, 0.0)

        for s in range(N_DEV - 1):
            send_b = (my - s) % N_DEV
            rdma = pltpu.make_async_remote_copy(
                src_ref=out_ref.at[pl.ds(send_b * m_blk, m_blk), :],
                dst_ref=out_ref.at[pl.ds(send_b * m_blk, m_blk), :],
                send_sem=ag_send_sems.at[s],
                recv_sem=ag_recv_sems.at[s],
                device_id=(right,),
                device_id_type=pl.DeviceIdType.MESH,
            )
            rdma.start()
            rdma.wait()

    return pl.pallas_call(
        body,
        out_shape=jax.ShapeDtypeStruct((m, n), jnp.float32),
        in_specs=[
            pl.BlockSpec(memory_space=pltpu.VMEM),
            pl.BlockSpec(memory_space=pltpu.VMEM),
            pl.BlockSpec(memory_space=pltpu.SMEM),
            pl.BlockSpec(memory_space=pltpu.SMEM),
        ],
        out_specs=pl.BlockSpec(memory_space=pltpu.VMEM),
        scratch_shapes=[
            pltpu.VMEM((N_DEV - 1, m_blk, n), jnp.float32),
            pltpu.SemaphoreType.DMA((N_DEV - 1,)),
            pltpu.SemaphoreType.DMA((N_DEV - 1,)),
            pltpu.SemaphoreType.DMA((N_DEV - 1,)),
            pltpu.SemaphoreType.DMA((N_DEV - 1,)),
        ],
        compiler_params=pltpu.CompilerParams(collective_id=0),
    )(x, w_mat, scale_x, scale_w)


# baseline (device time: 189344 ns/iter reference)
import jax
import jax.numpy as jnp
from jax import lax
from jax.experimental import pallas as pl
from jax.experimental.pallas import tpu as pltpu

N_DEV = 4
F32 = jnp.float32
BF16 = jnp.bfloat16


def kernel(x, w_mat, scale_x, scale_w):
    m, k_shard = x.shape
    _, n = w_mat.shape
    m_blk = m // N_DEV
    n_half = n // 2

    def body(x_hbm, w_ref, sx_ref, sw_ref, out_hbm,
             xb, stage, comm, load_sems, store_sems,
             rs_send, rs_recv, ag_send, ag_recv):
        my = lax.axis_index("i")
        left = (my - 1) % N_DEV
        right = (my + 1) % N_DEV
        nbr = [right, left]
        cols = [pl.ds(0, n_half), pl.ds(n_half, n_half)]

        barrier_sem = pltpu.get_barrier_semaphore()
        for d in range(2):
            pl.semaphore_signal(
                barrier_sem, inc=1,
                device_id=(nbr[d],), device_id_type=pl.DeviceIdType.MESH,
            )
        pl.semaphore_wait(barrier_sem, 2)

        wb_r = w_ref[:, :n_half].astype(BF16)
        wb_l = w_ref[:, n_half:].astype(BF16)

        def start_load(b, slot, sem_i):
            cp = pltpu.make_async_copy(
                x_hbm.at[pl.ds(b * m_blk, m_blk), :], xb.at[slot],
                load_sems.at[sem_i],
            )
            cp.start()
            return cp

        ld_a = start_load((my - 1) % N_DEV, 0, 0)
        ld_b = start_load((my + 1) % N_DEV, 1, 1)
        ld_c = start_load((my + 2) % N_DEV, 2, 2)

        ld_a.wait()
        xa = xb[0].astype(BF16)
        stage[0, 0] = jnp.dot(xa, wb_r, preferred_element_type=F32).astype(BF16)
        ld_b.wait()
        xbb = xb[1].astype(BF16)
        stage[1, 0] = jnp.dot(xbb, wb_l, preferred_element_type=F32).astype(BF16)

        scale = sx_ref[0] * sw_ref[0]

        for s in range(N_DEV - 1):
            slot = s % 2
            rdmas = []
            for d in range(2):
                r = pltpu.make_async_remote_copy(
                    src_ref=stage.at[d, slot],
                    dst_ref=comm.at[d, slot],
                    send_sem=rs_send.at[s, d],
                    recv_sem=rs_recv.at[s, d],
                    device_id=(nbr[d],),
                    device_id_type=pl.DeviceIdType.MESH,
                )
                r.start()
                rdmas.append(r)
            if s == 0:
                ld_c.wait()
                xc = xb[2].astype(BF16)
                p_r = jnp.dot(xc, wb_r, preferred_element_type=F32)
                p_l = jnp.dot(xc, wb_l, preferred_element_type=F32)
                ld_d = start_load(my, 2, 3)
            elif s == 1:
                p_r = jnp.dot(xbb, wb_r, preferred_element_type=F32)
                p_l = jnp.dot(xa, wb_l, preferred_element_type=F32)
            else:
                ld_d.wait()
                xd = xb[2].astype(BF16)
                p_r = jnp.dot(xd, wb_r, preferred_element_type=F32)
                p_l = jnp.dot(xd, wb_l, preferred_element_type=F32)
            for r in rdmas:
                r.wait()
            if s < N_DEV - 2:
                stage[0, 1 - slot] = (p_r + comm[0, slot].astype(F32)).astype(BF16)
                stage[1, 1 - slot] = (p_l + comm[1, slot].astype(F32)).astype(BF16)
            else:
                f_r = jnp.maximum((p_r + comm[0, slot].astype(F32)) * scale, 0.0)
                f_l = jnp.maximum((p_l + comm[1, slot].astype(F32)) * scale, 0.0)
                stage[0, 1 - slot] = f_r.astype(BF16)
                stage[1, 1 - slot] = f_l.astype(BF16)

        own_rows = pl.ds(my * m_blk, m_blk)
        own_stores = []
        for d in range(2):
            st = pltpu.make_async_copy(
                stage.at[d, 1], out_hbm.at[own_rows, cols[d]], store_sems.at[d],
            )
            st.start()
            own_stores.append(st)

        pending_stores = [None, None]
        for s in range(N_DEV - 1):
            slot = s % 2
            if pending_stores[slot] is not None:
                for st in pending_stores[slot]:
                    st.wait()
            rdmas = []
            for d in range(2):
                src = stage.at[d, 1] if s == 0 else comm.at[d, 1 - slot]
                r = pltpu.make_async_remote_copy(
                    src_ref=src,
                    dst_ref=comm.at[d, slot],
                    send_sem=ag_send.at[s, d],
                    recv_sem=ag_recv.at[s, d],
                    device_id=(nbr[d],),
                    device_id_type=pl.DeviceIdType.MESH,
                )
                r.start()
                rdmas.append(r)
            for r in rdmas:
                r.wait()
            recv_b = [(my - 1 - s) % N_DEV, (my + 1 + s) % N_DEV]
            stores = []
            for d in range(2):
                st = pltpu.make_async_copy(
                    comm.at[d, slot],
                    out_hbm.at[pl.ds(recv_b[d] * m_blk, m_blk), cols[d]],
                    store_sems.at[2 + 2 * s + d],
                )
                st.start()
                stores.append(st)
            pending_stores[slot] = stores

        for st in own_stores:
            st.wait()
        for stores in pending_stores:
            if stores is not None:
                for st in stores:
                    st.wait()

    return pl.pallas_call(
        body,
        out_shape=jax.ShapeDtypeStruct((m, n), BF16),
        in_specs=[
            pl.BlockSpec(memory_space=pl.ANY),
            pl.BlockSpec(memory_space=pltpu.VMEM),
            pl.BlockSpec(memory_space=pltpu.SMEM),
            pl.BlockSpec(memory_space=pltpu.SMEM),
        ],
        out_specs=pl.BlockSpec(memory_space=pl.ANY),
        scratch_shapes=[
            pltpu.VMEM((3, m_blk, k_shard), F32),
            pltpu.VMEM((2, 2, m_blk, n_half), BF16),
            pltpu.VMEM((2, 2, m_blk, n_half), BF16),
            pltpu.SemaphoreType.DMA((4,)),
            pltpu.SemaphoreType.DMA((8,)),
            pltpu.SemaphoreType.DMA((N_DEV - 1, 2)),
            pltpu.SemaphoreType.DMA((N_DEV - 1, 2)),
            pltpu.SemaphoreType.DMA((N_DEV - 1, 2)),
            pltpu.SemaphoreType.DMA((N_DEV - 1, 2)),
        ],
        compiler_params=pltpu.CompilerParams(
            collective_id=0, vmem_limit_bytes=64 * 1024 * 1024,
        ),
    )(x, w_mat, scale_x, scale_w)


# device time: 189088 ns/iter; 1.0014x vs baseline; 1.0014x over previous
import jax
import jax.numpy as jnp
from jax import lax
from jax.experimental import pallas as pl
from jax.experimental.pallas import tpu as pltpu

N_DEV = 4
F32 = jnp.float32
BF16 = jnp.bfloat16


def kernel(x, w_mat, scale_x, scale_w):
    m, k_shard = x.shape
    _, n = w_mat.shape
    m_blk = m // N_DEV
    n_half = n // 2

    def body(x_hbm, w_ref, sx_ref, sw_ref, out_hbm,
             xb, stage, comm, load_sems, store_sems,
             rs_send, rs_recv, ag_send, ag_recv):
        my = lax.axis_index("i")
        left = (my - 1) % N_DEV
        right = (my + 1) % N_DEV
        nbr = [right, left]
        cols = [pl.ds(0, n_half), pl.ds(n_half, n_half)]

        barrier_sem = pltpu.get_barrier_semaphore()
        for d in range(2):
            pl.semaphore_signal(
                barrier_sem, inc=1,
                device_id=(nbr[d],), device_id_type=pl.DeviceIdType.MESH,
            )
        pl.semaphore_wait(barrier_sem, 2)

        wb_r = w_ref[:, :n_half].astype(BF16)
        wb_l = w_ref[:, n_half:].astype(BF16)

        def start_load(b, slot, sem_i):
            cp = pltpu.make_async_copy(
                x_hbm.at[pl.ds(b * m_blk, m_blk), :], xb.at[slot],
                load_sems.at[sem_i],
            )
            cp.start()
            return cp

        ld_a = start_load((my - 1) % N_DEV, 0, 0)
        ld_b = start_load((my + 1) % N_DEV, 1, 1)
        ld_c = start_load((my + 2) % N_DEV, 2, 2)

        def mk_rs(s, d):
            return pltpu.make_async_remote_copy(
                src_ref=stage.at[d, s % 2],
                dst_ref=comm.at[d, s % 2],
                send_sem=rs_send.at[s, d],
                recv_sem=rs_recv.at[s, d],
                device_id=(nbr[d],),
                device_id_type=pl.DeviceIdType.MESH,
            )

        ld_a.wait()
        xa = xb[0].astype(BF16)
        stage[0, 0] = jnp.dot(xa, wb_r, preferred_element_type=F32).astype(BF16)
        rs0 = [mk_rs(0, 0)]
        rs0[0].start()
        ld_b.wait()
        xbb = xb[1].astype(BF16)
        stage[1, 0] = jnp.dot(xbb, wb_l, preferred_element_type=F32).astype(BF16)
        rs0.append(mk_rs(0, 1))
        rs0[1].start()

        scale = sx_ref[0] * sw_ref[0]

        for s in range(N_DEV - 1):
            slot = s % 2
            if s == 0:
                rdmas = rs0
            else:
                rdmas = []
                for d in range(2):
                    r = mk_rs(s, d)
                    r.start()
                    rdmas.append(r)
            if s == 0:
                ld_c.wait()
                xc = xb[2].astype(BF16)
                p_r = jnp.dot(xc, wb_r, preferred_element_type=F32)
                p_l = jnp.dot(xc, wb_l, preferred_element_type=F32)
                ld_d = start_load(my, 2, 3)
            elif s == 1:
                p_r = jnp.dot(xbb, wb_r, preferred_element_type=F32)
                p_l = jnp.dot(xa, wb_l, preferred_element_type=F32)
            else:
                ld_d.wait()
                xd = xb[2].astype(BF16)
                p_r = jnp.dot(xd, wb_r, preferred_element_type=F32)
                p_l = jnp.dot(xd, wb_l, preferred_element_type=F32)
            for r in rdmas:
                r.wait()
            if s < N_DEV - 2:
                stage[0, 1 - slot] = (p_r + comm[0, slot].astype(F32)).astype(BF16)
                stage[1, 1 - slot] = (p_l + comm[1, slot].astype(F32)).astype(BF16)
            else:
                f_r = jnp.maximum((p_r + comm[0, slot].astype(F32)) * scale, 0.0)
                f_l = jnp.maximum((p_l + comm[1, slot].astype(F32)) * scale, 0.0)
                stage[0, 1 - slot] = f_r.astype(BF16)
                stage[1, 1 - slot] = f_l.astype(BF16)
                ag0 = []
                for d in range(2):
                    r = pltpu.make_async_remote_copy(
                        src_ref=stage.at[d, 1],
                        dst_ref=comm.at[d, 0],
                        send_sem=ag_send.at[0, d],
                        recv_sem=ag_recv.at[0, d],
                        device_id=(nbr[d],),
                        device_id_type=pl.DeviceIdType.MESH,
                    )
                    r.start()
                    ag0.append(r)

        own_rows = pl.ds(my * m_blk, m_blk)
        own_stores = []
        for d in range(2):
            st = pltpu.make_async_copy(
                stage.at[d, 1], out_hbm.at[own_rows, cols[d]], store_sems.at[d],
            )
            st.start()
            own_stores.append(st)

        pending_stores = [None, None]
        for s in range(N_DEV - 1):
            slot = s % 2
            if pending_stores[slot] is not None:
                for st in pending_stores[slot]:
                    st.wait()
            if s == 0:
                rdmas = ag0
            else:
                rdmas = []
                for d in range(2):
                    r = pltpu.make_async_remote_copy(
                        src_ref=comm.at[d, 1 - slot],
                        dst_ref=comm.at[d, slot],
                        send_sem=ag_send.at[s, d],
                        recv_sem=ag_recv.at[s, d],
                        device_id=(nbr[d],),
                        device_id_type=pl.DeviceIdType.MESH,
                    )
                    r.start()
                    rdmas.append(r)
            for r in rdmas:
                r.wait()
            recv_b = [(my - 1 - s) % N_DEV, (my + 1 + s) % N_DEV]
            stores = []
            for d in range(2):
                st = pltpu.make_async_copy(
                    comm.at[d, slot],
                    out_hbm.at[pl.ds(recv_b[d] * m_blk, m_blk), cols[d]],
                    store_sems.at[2 + 2 * s + d],
                )
                st.start()
                stores.append(st)
            pending_stores[slot] = stores

        for st in own_stores:
            st.wait()
        for stores in pending_stores:
            if stores is not None:
                for st in stores:
                    st.wait()

    return pl.pallas_call(
        body,
        out_shape=jax.ShapeDtypeStruct((m, n), BF16),
        in_specs=[
            pl.BlockSpec(memory_space=pl.ANY),
            pl.BlockSpec(memory_space=pltpu.VMEM),
            pl.BlockSpec(memory_space=pltpu.SMEM),
            pl.BlockSpec(memory_space=pltpu.SMEM),
        ],
        out_specs=pl.BlockSpec(memory_space=pl.ANY),
        scratch_shapes=[
            pltpu.VMEM((3, m_blk, k_shard), F32),
            pltpu.VMEM((2, 2, m_blk, n_half), BF16),
            pltpu.VMEM((2, 2, m_blk, n_half), BF16),
            pltpu.SemaphoreType.DMA((4,)),
            pltpu.SemaphoreType.DMA((8,)),
            pltpu.SemaphoreType.DMA((N_DEV - 1, 2)),
            pltpu.SemaphoreType.DMA((N_DEV - 1, 2)),
            pltpu.SemaphoreType.DMA((N_DEV - 1, 2)),
            pltpu.SemaphoreType.DMA((N_DEV - 1, 2)),
        ],
        compiler_params=pltpu.CompilerParams(
            collective_id=0, vmem_limit_bytes=64 * 1024 * 1024,
        ),
    )(x, w_mat, scale_x, scale_w)


# device time: 148749 ns/iter; 1.2729x vs baseline; 1.2712x over previous
import jax
import jax.numpy as jnp
from jax import lax
from jax.experimental import pallas as pl
from jax.experimental.pallas import tpu as pltpu

N_DEV = 4
F32 = jnp.float32
BF16 = jnp.bfloat16
E4 = jnp.float8_e4m3fn
FP8_DOT = True


def kernel(x, w_mat, scale_x, scale_w):
    m, k_shard = x.shape
    _, n = w_mat.shape
    m_t = m // N_DEV

    def body(x_hbm, w_hbm, sx_ref, sw_ref, out_hbm,
             gA, gB, acc, temp_x, temp_w,
             ld_sems, s1_send, s1_recv, s2_send, s2_recv, store_sems):
        my = lax.axis_index("i")
        left = (my - 1) % N_DEV
        right = (my + 1) % N_DEV
        nbr = [right, left]

        x_lds = []
        for i in range(4):
            ld = pltpu.make_async_copy(
                x_hbm.at[pl.ds(i * 1024, 1024), :], temp_x,
                ld_sems.at[i])
            if i == 0:
                ld.start()
            x_lds.append(ld)
        w_lds = []
        for cw in range(2):
            ld = pltpu.make_async_copy(
                w_hbm.at[pl.ds(cw * 512, 512), :], temp_w, ld_sems.at[4 + cw])
            w_lds.append(ld)

        barrier_sem = pltpu.get_barrier_semaphore()
        for d in range(2):
            pl.semaphore_signal(
                barrier_sem, inc=1,
                device_id=(nbr[d],), device_id_type=pl.DeviceIdType.MESH,
            )
        pl.semaphore_wait(barrier_sem, 2)

        for i in range(4):
            x_lds[i].wait()
            if i < 3:
                gA[0, i] = temp_x[:, :].astype(E4)
            else:
                gB[0, 0] = temp_x[:, :].astype(E4)
            if i + 1 < 4:
                x_lds[i + 1].start()
            if i == 0:
                w_lds[0].start()

        def mk_s1(buf, h, d):
            return pltpu.make_async_remote_copy(
                src_ref=buf.at[0],
                dst_ref=buf.at[1 + d],
                send_sem=s1_send.at[h, d],
                recv_sem=s1_recv.at[h, d],
                device_id=(nbr[d],),
                device_id_type=pl.DeviceIdType.MESH,
            )

        s1a = [mk_s1(gA, 0, d) for d in range(2)]
        for r in s1a:
            r.start()

        for cw in range(2):
            w_lds[cw].wait()
            gB[0, 1 + cw] = temp_w[:, :].astype(E4).reshape(1024, 1024)
            if cw == 0:
                w_lds[1].start()

        def panel_w(slot):
            wflat = jnp.concatenate([gB[slot, 1], gB[slot, 2]], axis=0)
            wmat = wflat.reshape(k_shard, n)
            return wmat if FP8_DOT else wmat.astype(BF16)

        def x_tile(slot, mt):
            xt = gA[slot, mt] if mt < 3 else gB[slot, 0]
            return xt if FP8_DOT else xt.astype(BF16)

        def panel_dots(slot, first=False):
            wmat = panel_w(slot)
            for mt in range(N_DEV):
                dt = jnp.dot(x_tile(slot, mt), wmat,
                             preferred_element_type=F32)
                if first:
                    acc[mt] = dt.astype(BF16)
                else:
                    acc[mt] = (acc[mt].astype(F32) + dt).astype(BF16)

        panel_dots(0, first=True)
        for r in s1a:
            r.wait_send()
        s1b = [mk_s1(gB, 1, d) for d in range(2)]
        for r in s1b:
            r.start()
        for r in s1a:
            r.wait_recv()
        for r in s1b:
            r.wait()

        fwd = [(gA, 1), (gB, 2)]
        s2 = []
        for d in range(2):
            buf, src_slot = fwd[d]
            r = pltpu.make_async_remote_copy(
                src_ref=buf.at[src_slot],
                dst_ref=buf.at[3],
                send_sem=s2_send.at[d],
                recv_sem=s2_recv.at[d],
                device_id=(nbr[d],),
                device_id_type=pl.DeviceIdType.MESH,
            )
            r.start()
            s2.append(r)

        panel_dots(1)
        panel_dots(2)
        for r in s2:
            r.wait()

        scale = sx_ref[0] * sw_ref[0]
        wmat = panel_w(3)
        stores = []
        for mt in range(N_DEV):
            dt = jnp.dot(x_tile(3, mt), wmat, preferred_element_type=F32)
            fin = jnp.maximum((acc[mt].astype(F32) + dt) * scale, 0.0)
            acc[mt] = fin.astype(BF16)
            st = pltpu.make_async_copy(
                acc.at[mt], out_hbm.at[pl.ds(mt * m_t, m_t), :],
                store_sems.at[mt],
            )
            st.start()
            stores.append(st)
        for st in stores:
            st.wait()

    return pl.pallas_call(
        body,
        out_shape=jax.ShapeDtypeStruct((m, n), BF16),
        in_specs=[
            pl.BlockSpec(memory_space=pl.ANY),
            pl.BlockSpec(memory_space=pl.ANY),
            pl.BlockSpec(memory_space=pltpu.SMEM),
            pl.BlockSpec(memory_space=pltpu.SMEM),
        ],
        out_specs=pl.BlockSpec(memory_space=pl.ANY),
        scratch_shapes=[
            pltpu.VMEM((N_DEV, 3, 1024, 1024), E4),
            pltpu.VMEM((N_DEV, 3, 1024, 1024), E4),
            pltpu.VMEM((N_DEV, m_t, n), BF16),
            pltpu.VMEM((1024, k_shard), F32),
            pltpu.VMEM((512, n), F32),
            pltpu.SemaphoreType.DMA((6,)),
            pltpu.SemaphoreType.DMA((2, 2)),
            pltpu.SemaphoreType.DMA((2, 2)),
            pltpu.SemaphoreType.DMA((2,)),
            pltpu.SemaphoreType.DMA((2,)),
            pltpu.SemaphoreType.DMA((N_DEV,)),
        ],
        compiler_params=pltpu.CompilerParams(
            collective_id=0, vmem_limit_bytes=64 * 1024 * 1024,
        ),
    )(x, w_mat, scale_x, scale_w)
